# baseline (speedup 1.0000x reference)
import math

import jax
import jax.numpy as jnp
import numpy as np
from jax import lax
from jax.experimental import pallas as pl
from jax.experimental.pallas import tpu as pltpu
from jax.experimental.pallas import tpu_sc as plsc

VOCAB = 100000
S_LAB = 35
MAX_CARD = 256
D = 128
B = 1024
HIST = 200
CUR = 50

NC = 2
NS = 16
NW = NC * NS

N_HIST = B * HIST
N_CUR = B * CUR
N_ROWS = N_HIST + N_CUR
RPT = N_ROWS // NW
CHUNK = 200
CPT = RPT // CHUNK
HIST_TILES = N_HIST // RPT
HIST_REM = N_HIST - HIST_TILES * RPT


def _positional_encoding(seq_len, d_model):
    pos = np.arange(seq_len, dtype=np.float32)[:, None]
    div = np.exp(np.arange(0, d_model, 2, dtype=np.float32) * (-math.log(10000.0) / d_model))
    pe = np.zeros((seq_len, d_model), dtype=np.float32)
    pe[:, 0::2] = np.sin(pos * div)
    pe[:, 1::2] = np.cos(pos * div)
    return pe


_PE_NP = _positional_encoding(HIST, D)


def _sc_gather_body(ph_hbm, pc_hbm, table_hbm, combo_hbm,
                    hist_out, cur_out,
                    pv, ti0, ci0, ti1, ci1, rt0, rc0, rt1, rc1, combo_sp,
                    sem_t0, sem_c0, sem_t1, sem_c1, sem_s0, sem_s1):
    wid = lax.axis_index("s") * NC + lax.axis_index("c")
    row0 = wid * RPT

    @pl.when(lax.axis_index("s") == 0)
    def _():
        pltpu.sync_copy(combo_hbm, combo_sp)

    plsc.subcore_barrier()

    @pl.when(wid < HIST_TILES)
    def _():
        pltpu.sync_copy(ph_hbm.at[pl.ds(row0, RPT)], pv.at[pl.ds(0, RPT)])

    @pl.when(wid == HIST_TILES)
    def _():
        pltpu.sync_copy(ph_hbm.at[pl.ds(HIST_TILES * RPT, HIST_REM)], pv.at[pl.ds(0, HIST_REM)])
        pltpu.sync_copy(pc_hbm.at[pl.ds(0, RPT - HIST_REM)], pv.at[pl.ds(HIST_REM, RPT - HIST_REM)])

    @pl.when(wid > HIST_TILES)
    def _():
        pltpu.sync_copy(pc_hbm.at[pl.ds(row0 - N_HIST, RPT)], pv.at[pl.ds(0, RPT)])

    iota16 = lax.broadcasted_iota(jnp.int32, (16,), 0)

    def unpack_chunk(c, ti, ci):
        def ub(i, carry):
            sl16 = pl.ds(16 * i, 16)
            p = pv[pl.ds(c * CHUNK + 16 * i, 16)].astype(jnp.int32)
            tid = lax.bitwise_and(p, 0x000FFFFF)
            role = lax.shift_right_logical(p, 20)
            n = row0 + c * CHUNK + 16 * i + iota16
            ch = role * HIST + lax.rem(n, HIST)
            cc2 = 4 * HIST + role * CUR + lax.rem(n - N_HIST, CUR)
            ti[sl16] = tid
            ci[sl16] = jnp.where(n < N_HIST, ch, cc2)
            return carry

        lax.fori_loop(0, 13, ub, 0)

    def fire_gathers(ti, ci, rt, rc, sem_t, sem_c):
        pltpu.async_copy(table_hbm.at[ti.at[pl.ds(0, 96)]], rt.at[pl.ds(0, 96)], sem_t)
        pltpu.async_copy(table_hbm.at[ti.at[pl.ds(96, 104)]], rt.at[pl.ds(96, 104)], sem_t)
        pltpu.async_copy(combo_sp.at[ci.at[pl.ds(0, 96)]], rc.at[pl.ds(0, 96)], sem_c)
        pltpu.async_copy(combo_sp.at[ci.at[pl.ds(96, 104)]], rc.at[pl.ds(96, 104)], sem_c)

    def wait_gathers(ti, ci, rt, rc, sem_t, sem_c):
        pltpu.make_async_copy(table_hbm.at[ti.at[pl.ds(0, 96)]], rt.at[pl.ds(0, 96)], sem_t).wait()
        pltpu.make_async_copy(table_hbm.at[ti.at[pl.ds(96, 104)]], rt.at[pl.ds(96, 104)], sem_t).wait()
        pltpu.make_async_copy(combo_sp.at[ci.at[pl.ds(0, 96)]], rc.at[pl.ds(0, 96)], sem_c).wait()
        pltpu.make_async_copy(combo_sp.at[ci.at[pl.ds(96, 104)]], rc.at[pl.ds(96, 104)], sem_c).wait()

    def fire_store(c, rt, sem_s):
        base = row0 + c * CHUNK

        @pl.when(base < N_HIST)
        def _():
            pltpu.async_copy(rt, hist_out.at[base // HIST], sem_s)

        @pl.when(base >= N_HIST)
        def _():
            cb = (base - N_HIST) // CUR
            for q in range(CHUNK // CUR):
                pltpu.async_copy(rt.at[pl.ds(CUR * q, CUR)], cur_out.at[cb + q], sem_s)

    def wait_store(rt, sem_s):
        pltpu.make_async_copy(rt, hist_out.at[0], sem_s).wait()

    def add_rows(rt, rc):
        def add_row(r, carry):
            for j in range(D // 16):
                sl = pl.ds(16 * j, 16)
                rt[r, sl] = rt[r, sl] + rc[r, sl]
            return carry

        lax.fori_loop(0, CHUNK, add_row, 0)

    unpack_chunk(0, ti0, ci0)
    fire_gathers(ti0, ci0, rt0, rc0, sem_t0, sem_c0)
    unpack_chunk(1, ti1, ci1)
    fire_gathers(ti1, ci1, rt1, rc1, sem_t1, sem_c1)

    def pair_body(k, carry):
        c = 2 * k
        wait_gathers(ti0, ci0, rt0, rc0, sem_t0, sem_c0)
        add_rows(rt0, rc0)
        fire_store(c, rt0, sem_s0)

        wait_gathers(ti1, ci1, rt1, rc1, sem_t1, sem_c1)
        add_rows(rt1, rc1)
        fire_store(c + 1, rt1, sem_s1)

        @pl.when(c + 2 < CPT)
        def _():
            wait_store(rt0, sem_s0)
            unpack_chunk(c + 2, ti0, ci0)
            fire_gathers(ti0, ci0, rt0, rc0, sem_t0, sem_c0)

        @pl.when(c + 3 < CPT)
        def _():
            wait_store(rt1, sem_s1)
            unpack_chunk(c + 3, ti1, ci1)
            fire_gathers(ti1, ci1, rt1, rc1, sem_t1, sem_c1)

        return carry

    lax.fori_loop(0, CPT // 2, pair_body, 0)
    wait_store(rt0, sem_s0)
    wait_store(rt1, sem_s1)


_sc_gather = pl.kernel(
    _sc_gather_body,
    out_type=[
        jax.ShapeDtypeStruct((B, HIST, D), jnp.float32),
        jax.ShapeDtypeStruct((B, CUR, D), jnp.float32),
    ],
    mesh=plsc.VectorSubcoreMesh(core_axis_name="c", subcore_axis_name="s"),
    scratch_types=[
        pltpu.VMEM((RPT + 16,), jnp.float32),
        pltpu.VMEM((CHUNK + 8,), jnp.int32),
        pltpu.VMEM((CHUNK + 8,), jnp.int32),
        pltpu.VMEM((CHUNK + 8,), jnp.int32),
        pltpu.VMEM((CHUNK + 8,), jnp.int32),
        pltpu.VMEM((CHUNK, D), jnp.float32),
        pltpu.VMEM((CHUNK, D), jnp.float32),
        pltpu.VMEM((CHUNK, D), jnp.float32),
        pltpu.VMEM((CHUNK, D), jnp.float32),
        pltpu.VMEM_SHARED((4 * HIST + 4 * CUR, D), jnp.float32),
        pltpu.SemaphoreType.DMA,
        pltpu.SemaphoreType.DMA,
        pltpu.SemaphoreType.DMA,
        pltpu.SemaphoreType.DMA,
        pltpu.SemaphoreType.DMA,
        pltpu.SemaphoreType.DMA,
    ],
)


def _state_mm_body(prev_hbm, cur_hbm, emb_hbm, outp_hbm, outc_hbm,
                   emb_v, pb0, cb0, pb1, cb1, ob0, oc0, ob1, oc1,
                   sem_e, sem_in, sem_out):
    ecopy = pltpu.async_copy(emb_hbm, emb_v, sem_e)

    def fire_in(b, pb, cb):
        pltpu.async_copy(prev_hbm.at[pl.ds(b * _BB, _BB)], pb, sem_in)
        pltpu.async_copy(cur_hbm.at[pl.ds(b * _BB, _BB)], cb, sem_in)

    def wait_in(b, pb, cb):
        pltpu.make_async_copy(prev_hbm.at[pl.ds(b * _BB, _BB)], pb, sem_in).wait()
        pltpu.make_async_copy(cur_hbm.at[pl.ds(b * _BB, _BB)], cb, sem_in).wait()

    def wait_out(b, ob, oc):
        pltpu.make_async_copy(ob, outp_hbm.at[pl.ds(b * _BB, _BB)], sem_out).wait()
        pltpu.make_async_copy(oc, outc_hbm.at[pl.ds(b * _BB, _BB)], sem_out).wait()

    bufs = [(pb0, cb0, ob0, oc0), (pb1, cb1, ob1, oc1)]
    fire_in(0, pb0, cb0)
    ecopy.wait()
    nblk = B // _BB
    for b in range(nblk):
        pb, cb, ob, oc = bufs[b % 2]
        if b + 1 < nblk:
            fire_in(b + 1, *bufs[(b + 1) % 2][:2])
        wait_in(b, pb, cb)
        if b >= 2:
            wait_out(b - 2, ob, oc)
        for i in range(S_LAB):
            e = emb_v[i]
            ob[:, i, :] = jnp.dot(pb[:, i, :], e, preferred_element_type=jnp.float32)
            oc[:, i, :] = jnp.dot(cb[:, i, :], e, preferred_element_type=jnp.float32)
        pltpu.async_copy(ob, outp_hbm.at[pl.ds(b * _BB, _BB)], sem_out)
        pltpu.async_copy(oc, outc_hbm.at[pl.ds(b * _BB, _BB)], sem_out)
    wait_out(nblk - 2, *bufs[(nblk - 2) % 2][2:])
    wait_out(nblk - 1, *bufs[(nblk - 1) % 2][2:])


_BB = 128

_state_mm = pl.pallas_call(
    _state_mm_body,
    in_specs=[
        pl.BlockSpec(memory_space=pl.ANY),
        pl.BlockSpec(memory_space=pl.ANY),
        pl.BlockSpec(memory_space=pl.ANY),
    ],
    out_specs=[
        pl.BlockSpec(memory_space=pl.ANY),
        pl.BlockSpec(memory_space=pl.ANY),
    ],
    out_shape=[
        jax.ShapeDtypeStruct((B, S_LAB, D), jnp.float32),
        jax.ShapeDtypeStruct((B, S_LAB, D), jnp.float32),
    ],
    scratch_shapes=[
        pltpu.VMEM((S_LAB, MAX_CARD, D), jnp.float32),
        pltpu.VMEM((_BB, S_LAB, MAX_CARD), jnp.float32),
        pltpu.VMEM((_BB, S_LAB, MAX_CARD), jnp.float32),
        pltpu.VMEM((_BB, S_LAB, MAX_CARD), jnp.float32),
        pltpu.VMEM((_BB, S_LAB, MAX_CARD), jnp.float32),
        pltpu.VMEM((_BB, S_LAB, D), jnp.float32),
        pltpu.VMEM((_BB, S_LAB, D), jnp.float32),
        pltpu.VMEM((_BB, S_LAB, D), jnp.float32),
        pltpu.VMEM((_BB, S_LAB, D), jnp.float32),
        pltpu.SemaphoreType.DMA,
        pltpu.SemaphoreType.DMA,
        pltpu.SemaphoreType.DMA,
    ],
)


def kernel(previous_state, current_state, history_text, current_text,
           history_roles, current_roles, text_table, state_embedding):
    pe = jnp.asarray(_PE_NP)
    t4 = text_table[:4]
    combo_h = (t4[:, None, :] + 2.0 * pe[None, :, :]).reshape(4 * HIST, D)
    combo_c = (t4[:, None, :] + 2.0 * pe[None, :CUR, :]).reshape(4 * CUR, D)
    combo = jnp.concatenate([combo_h, combo_c], axis=0)

    ph = (history_text | (history_roles << 20)).reshape(-1).astype(jnp.float32)
    pc = (current_text | (current_roles << 20)).reshape(-1).astype(jnp.float32)

    hist_text_embed, curr_text_embed = _sc_gather(ph, pc, text_table, combo)

    pre_state_embed, cur_state_embed = _state_mm(
        previous_state, current_state, state_embedding)

    return (pre_state_embed, cur_state_embed, hist_text_embed, curr_text_embed)


# speedup vs baseline: 1.0109x; 1.0109x over previous
import math

import jax
import jax.numpy as jnp
import numpy as np
from jax import lax
from jax.experimental import pallas as pl
from jax.experimental.pallas import tpu as pltpu
from jax.experimental.pallas import tpu_sc as plsc

VOCAB = 100000
S_LAB = 35
MAX_CARD = 256
D = 128
B = 1024
HIST = 200
CUR = 50

NC = 2
NS = 16
NW = NC * NS

N_HIST = B * HIST
N_CUR = B * CUR
N_ROWS = N_HIST + N_CUR
RPT = N_ROWS // NW
CHUNK = 200
CPT = RPT // CHUNK
HIST_TILES = N_HIST // RPT
HIST_REM = N_HIST - HIST_TILES * RPT


def _positional_encoding(seq_len, d_model):
    pos = np.arange(seq_len, dtype=np.float32)[:, None]
    div = np.exp(np.arange(0, d_model, 2, dtype=np.float32) * (-math.log(10000.0) / d_model))
    pe = np.zeros((seq_len, d_model), dtype=np.float32)
    pe[:, 0::2] = np.sin(pos * div)
    pe[:, 1::2] = np.cos(pos * div)
    return pe


_PE_NP = _positional_encoding(HIST, D)


def _sc_gather_body(ph_hbm, pc_hbm, table_hbm, combo_hbm,
                    hist_out, cur_out,
                    pv, ti0, ci0, ti1, ci1, rt0, rc0, rt1, rc1, combo_sp,
                    sem_t0, sem_c0, sem_t1, sem_c1, sem_s0, sem_s1):
    wid = lax.axis_index("s") * NC + lax.axis_index("c")
    row0 = wid * RPT

    @pl.when(lax.axis_index("s") == 0)
    def _():
        pltpu.sync_copy(combo_hbm, combo_sp)

    plsc.subcore_barrier()

    @pl.when(wid < HIST_TILES)
    def _():
        pltpu.sync_copy(ph_hbm.at[pl.ds(row0, RPT)], pv.at[pl.ds(0, RPT)])

    @pl.when(wid == HIST_TILES)
    def _():
        pltpu.sync_copy(ph_hbm.at[pl.ds(HIST_TILES * RPT, HIST_REM)], pv.at[pl.ds(0, HIST_REM)])
        pltpu.sync_copy(pc_hbm.at[pl.ds(0, RPT - HIST_REM)], pv.at[pl.ds(HIST_REM, RPT - HIST_REM)])

    @pl.when(wid > HIST_TILES)
    def _():
        pltpu.sync_copy(pc_hbm.at[pl.ds(row0 - N_HIST, RPT)], pv.at[pl.ds(0, RPT)])

    iota16 = lax.broadcasted_iota(jnp.int32, (16,), 0)

    def unpack_chunk(c, ti, ci):
        def ub(i, carry):
            sl16 = pl.ds(16 * i, 16)
            p = pv[pl.ds(c * CHUNK + 16 * i, 16)].astype(jnp.int32)
            tid = lax.bitwise_and(p, 0x000FFFFF)
            role = lax.shift_right_logical(p, 20)
            n = row0 + c * CHUNK + 16 * i + iota16
            ch = role * HIST + lax.rem(n, HIST)
            cc2 = 4 * HIST + role * CUR + lax.rem(n - N_HIST, CUR)
            ti[sl16] = tid
            ci[sl16] = jnp.where(n < N_HIST, ch, cc2)
            return carry

        lax.fori_loop(0, 13, ub, 0)

    def fire_gathers(ti, ci, rt, rc, sem_t, sem_c):
        pltpu.async_copy(table_hbm.at[ti.at[pl.ds(0, 96)]], rt.at[pl.ds(0, 96)], sem_t)
        pltpu.async_copy(table_hbm.at[ti.at[pl.ds(96, 104)]], rt.at[pl.ds(96, 104)], sem_t)
        pltpu.async_copy(combo_sp.at[ci.at[pl.ds(0, 96)]], rc.at[pl.ds(0, 96)], sem_c)
        pltpu.async_copy(combo_sp.at[ci.at[pl.ds(96, 104)]], rc.at[pl.ds(96, 104)], sem_c)

    def wait_gathers(ti, ci, rt, rc, sem_t, sem_c):
        pltpu.make_async_copy(table_hbm.at[ti.at[pl.ds(0, 96)]], rt.at[pl.ds(0, 96)], sem_t).wait()
        pltpu.make_async_copy(table_hbm.at[ti.at[pl.ds(96, 104)]], rt.at[pl.ds(96, 104)], sem_t).wait()
        pltpu.make_async_copy(combo_sp.at[ci.at[pl.ds(0, 96)]], rc.at[pl.ds(0, 96)], sem_c).wait()
        pltpu.make_async_copy(combo_sp.at[ci.at[pl.ds(96, 104)]], rc.at[pl.ds(96, 104)], sem_c).wait()

    def fire_store(c, rt, sem_s):
        base = row0 + c * CHUNK

        @pl.when(base < N_HIST)
        def _():
            pltpu.async_copy(rt, hist_out.at[base // HIST], sem_s)

        @pl.when(base >= N_HIST)
        def _():
            cb = (base - N_HIST) // CUR
            for q in range(CHUNK // CUR):
                pltpu.async_copy(rt.at[pl.ds(CUR * q, CUR)], cur_out.at[cb + q], sem_s)

    def wait_store(rt, sem_s):
        pltpu.make_async_copy(rt, hist_out.at[0], sem_s).wait()

    def add_rows(rt, rc):
        def add_row(r, carry):
            for j in range(D // 16):
                sl = pl.ds(16 * j, 16)
                rt[r, sl] = rt[r, sl] + rc[r, sl]
            return carry

        lax.fori_loop(0, CHUNK, add_row, 0)

    unpack_chunk(0, ti0, ci0)
    fire_gathers(ti0, ci0, rt0, rc0, sem_t0, sem_c0)
    unpack_chunk(1, ti1, ci1)
    fire_gathers(ti1, ci1, rt1, rc1, sem_t1, sem_c1)

    def pair_body(k, carry):
        c = 2 * k
        wait_gathers(ti0, ci0, rt0, rc0, sem_t0, sem_c0)
        add_rows(rt0, rc0)
        fire_store(c, rt0, sem_s0)

        wait_gathers(ti1, ci1, rt1, rc1, sem_t1, sem_c1)
        add_rows(rt1, rc1)
        fire_store(c + 1, rt1, sem_s1)

        @pl.when(c + 2 < CPT)
        def _():
            wait_store(rt0, sem_s0)
            unpack_chunk(c + 2, ti0, ci0)
            fire_gathers(ti0, ci0, rt0, rc0, sem_t0, sem_c0)

        @pl.when(c + 3 < CPT)
        def _():
            wait_store(rt1, sem_s1)
            unpack_chunk(c + 3, ti1, ci1)
            fire_gathers(ti1, ci1, rt1, rc1, sem_t1, sem_c1)

        return carry

    lax.fori_loop(0, CPT // 2, pair_body, 0)
    wait_store(rt0, sem_s0)
    wait_store(rt1, sem_s1)


_sc_gather = pl.kernel(
    _sc_gather_body,
    out_type=[
        jax.ShapeDtypeStruct((B, HIST, D), jnp.float32),
        jax.ShapeDtypeStruct((B, CUR, D), jnp.float32),
    ],
    mesh=plsc.VectorSubcoreMesh(core_axis_name="c", subcore_axis_name="s"),
    scratch_types=[
        pltpu.VMEM((RPT + 16,), jnp.float32),
        pltpu.VMEM((CHUNK + 8,), jnp.int32),
        pltpu.VMEM((CHUNK + 8,), jnp.int32),
        pltpu.VMEM((CHUNK + 8,), jnp.int32),
        pltpu.VMEM((CHUNK + 8,), jnp.int32),
        pltpu.VMEM((CHUNK, D), jnp.float32),
        pltpu.VMEM((CHUNK, D), jnp.float32),
        pltpu.VMEM((CHUNK, D), jnp.float32),
        pltpu.VMEM((CHUNK, D), jnp.float32),
        pltpu.VMEM_SHARED((4 * HIST + 4 * CUR, D), jnp.float32),
        pltpu.SemaphoreType.DMA,
        pltpu.SemaphoreType.DMA,
        pltpu.SemaphoreType.DMA,
        pltpu.SemaphoreType.DMA,
        pltpu.SemaphoreType.DMA,
        pltpu.SemaphoreType.DMA,
    ],
)


def _state_mm_body(prev_ref, cur_ref, emb_ref, outp_ref, outc_ref):
    for i in range(S_LAB):
        e = emb_ref[i]
        outp_ref[:, i, :] = jnp.dot(prev_ref[:, i, :], e, preferred_element_type=jnp.float32)
        outc_ref[:, i, :] = jnp.dot(cur_ref[:, i, :], e, preferred_element_type=jnp.float32)


_BB = 128

_state_mm = pl.pallas_call(
    _state_mm_body,
    grid=(B // _BB,),
    in_specs=[
        pl.BlockSpec((_BB, S_LAB, MAX_CARD), lambda b: (b, 0, 0)),
        pl.BlockSpec((_BB, S_LAB, MAX_CARD), lambda b: (b, 0, 0)),
        pl.BlockSpec((S_LAB, MAX_CARD, D), lambda b: (0, 0, 0)),
    ],
    out_specs=[
        pl.BlockSpec((_BB, S_LAB, D), lambda b: (b, 0, 0)),
        pl.BlockSpec((_BB, S_LAB, D), lambda b: (b, 0, 0)),
    ],
    out_shape=[
        jax.ShapeDtypeStruct((B, S_LAB, D), jnp.float32),
        jax.ShapeDtypeStruct((B, S_LAB, D), jnp.float32),
    ],
)


def kernel(previous_state, current_state, history_text, current_text,
           history_roles, current_roles, text_table, state_embedding):
    pe = jnp.asarray(_PE_NP)
    t4 = text_table[:4]
    combo_h = (t4[:, None, :] + 2.0 * pe[None, :, :]).reshape(4 * HIST, D)
    combo_c = (t4[:, None, :] + 2.0 * pe[None, :CUR, :]).reshape(4 * CUR, D)
    combo = jnp.concatenate([combo_h, combo_c], axis=0)

    ph = (history_text | (history_roles << 20)).reshape(-1).astype(jnp.float32)
    pc = (current_text | (current_roles << 20)).reshape(-1).astype(jnp.float32)

    hist_text_embed, curr_text_embed = _sc_gather(ph, pc, text_table, combo)

    pre_state_embed, cur_state_embed = _state_mm(
        previous_state, current_state, state_embedding)

    return (pre_state_embed, cur_state_embed, hist_text_embed, curr_text_embed)
